# baseline (speedup 1.0000x reference)
import functools

import jax
import jax.numpy as jnp
from jax import lax
from jax.experimental import pallas as pl
from jax.experimental.pallas import tpu as pltpu
from jax.experimental.pallas import tpu_sc as plsc

N = 10000
E = 320000
D = 128
ED = 16
G = 512
C = 20000
T_STEPS = 3
S2S_STEPS = 3
HID = 1024
YR0, YR1 = -36.2186, 204.88

F32 = jnp.float32
HIGH = lax.Precision.HIGHEST

NC = 2
NS = 16
NW = NC * NS
LANES = 16

TILE_ROWS = 632
LAST_ROWS = N - (NS - 1) * TILE_ROWS

EK = 128
N_ECHUNK = E // EK
ECHUNK_ITERS = -(-N_ECHUNK // NW)
EPAIRS = (ECHUNK_ITERS + 1) // 2

CK = 16
N_CCHUNK = C // CK
CKIDX = 48
CPAIRS = CKIDX // 2

NBLK = 1000
EBLK = 4000


def _pre_body(node_ref, wpre_ref, bpre_ref, wsrc_ref, wdst_ref,
              h_ref, hs_ref, hd_ref):
    h = jnp.maximum(
        jnp.dot(node_ref[...], wpre_ref[...], precision=lax.Precision.DEFAULT,
                preferred_element_type=F32) + bpre_ref[...], 0.0)
    h_ref[...] = h
    hs_ref[...] = jnp.dot(h, wsrc_ref[...], precision=HIGH,
                          preferred_element_type=F32)
    hd_ref[...] = jnp.dot(h, wdst_ref[...], precision=HIGH,
                          preferred_element_type=F32)


def _tc_pre(node, w_pre, b_pre, w_src, w_dst):
    grid = N // NBLK
    return pl.pallas_call(
        _pre_body,
        grid=(grid,),
        in_specs=[
            pl.BlockSpec((NBLK, D), lambda i: (i, 0)),
            pl.BlockSpec((D, D), lambda i: (0, 0)),
            pl.BlockSpec((1, D), lambda i: (0, 0)),
            pl.BlockSpec((D, D), lambda i: (0, 0)),
            pl.BlockSpec((D, D), lambda i: (0, 0)),
        ],
        out_specs=[
            pl.BlockSpec((NBLK, D), lambda i: (i, 0)),
            pl.BlockSpec((NBLK, D), lambda i: (i, 0)),
            pl.BlockSpec((NBLK, D), lambda i: (i, 0)),
        ],
        out_shape=[jax.ShapeDtypeStruct((N, D), F32)] * 3,
    )(node, w_pre, b_pre, w_src, w_dst)


def _eproj_body(edge_ref, we_ref, mb_ref, out_ref):
    out_ref[...] = jnp.dot(edge_ref[...], we_ref[...], precision=HIGH,
                           preferred_element_type=F32) + mb_ref[...]


def _tc_eproj(edge, w_e, msg_b):
    grid = E // EBLK
    return pl.pallas_call(
        _eproj_body,
        grid=(grid,),
        in_specs=[
            pl.BlockSpec((EBLK, ED), lambda i: (i, 0)),
            pl.BlockSpec((ED, D), lambda i: (0, 0)),
            pl.BlockSpec((1, D), lambda i: (0, 0)),
        ],
        out_specs=pl.BlockSpec((EBLK, D), lambda i: (i, 0)),
        out_shape=jax.ShapeDtypeStruct((E, D), F32),
    )(edge, w_e, msg_b)


def _sc_edge_body(hs_hbm, hd_hbm, eb_hbm, ei_hbm, zeros_hbm,
                  out_hbm,
                  sdv, buf_a, buf_b, buf_c, acc, sem_a, sem_b):
    cid = lax.axis_index("c")
    sid = lax.axis_index("s")
    wid = sid * NC + cid
    row_base = sid * TILE_ROWS

    @pl.when(sid < NS - 1)
    def _():
        pltpu.sync_copy(zeros_hbm, acc.at[pl.ds(row_base, TILE_ROWS)])

    @pl.when(sid == NS - 1)
    def _():
        pltpu.sync_copy(zeros_hbm.at[pl.ds(0, LAST_ROWS)],
                        acc.at[pl.ds(row_base, LAST_ROWS)])

    plsc.subcore_barrier()

    def chunk_body(k, carry):
        chunk = wid + k * NW

        @pl.when(chunk < N_ECHUNK)
        def _():
            base = chunk * EK
            pltpu.sync_copy(ei_hbm.at[:, pl.ds(base, EK)], sdv)
            cp_a = pltpu.async_copy(hs_hbm.at[sdv.at[0]], buf_a, sem_a)
            cp_b = pltpu.async_copy(hd_hbm.at[sdv.at[1]], buf_b, sem_b)
            pltpu.sync_copy(eb_hbm.at[pl.ds(base, EK)], buf_c)
            cp_a.wait()
            cp_b.wait()

            def row_body(i, c2):
                for col in range(D // LANES):
                    sl = pl.ds(col * LANES, LANES)
                    v = buf_a[i, sl] + buf_b[i, sl] + buf_c[i, sl]
                    buf_a[i, sl] = jnp.maximum(v, 0.0)
                return c2

            lax.fori_loop(0, EK, row_body, 0, unroll=4)
            pltpu.sync_copy(buf_a, acc.at[sdv.at[1]], add=True)

        return carry

    lax.fori_loop(0, ECHUNK_ITERS, chunk_body, 0, unroll=False)
    plsc.subcore_barrier()

    @pl.when(sid < NS - 1)
    def _():
        pltpu.sync_copy(acc.at[pl.ds(row_base, TILE_ROWS)],
                        out_hbm.at[cid, pl.ds(row_base, TILE_ROWS)])

    @pl.when(sid == NS - 1)
    def _():
        pltpu.sync_copy(acc.at[pl.ds(row_base, LAST_ROWS)],
                        out_hbm.at[cid, pl.ds(row_base, LAST_ROWS)])


def _sc_edge(hs, hd, eb, edge_index, zeros_tile):
    mesh = plsc.VectorSubcoreMesh(core_axis_name="c", subcore_axis_name="s")
    fn = pl.kernel(
        _sc_edge_body,
        out_type=jax.ShapeDtypeStruct((NC, N, D), F32),
        mesh=mesh,
        scratch_types=[
            pltpu.VMEM((2, EK), jnp.int32),
            pltpu.VMEM((EK, D), F32),
            pltpu.VMEM((EK, D), F32),
            pltpu.VMEM((EK, D), F32),
            pltpu.VMEM_SHARED((N, D), F32),
            pltpu.SemaphoreType.DMA,
            pltpu.SemaphoreType.DMA,
        ],
    )
    return fn(hs, hd, eb, edge_index, zeros_tile)


def _gru_body(p0_ref, p1_ref, h_ref, wi_ref, wh_ref, bi_ref, bh_ref,
              wsrc_ref, wdst_ref, hn_ref, hs_ref, hd_ref):
    m = p0_ref[0] + p1_ref[0]
    h = h_ref[...]
    gi = jnp.dot(m, wi_ref[...], precision=lax.Precision.DEFAULT,
                 preferred_element_type=F32) + bi_ref[...]
    gh = jnp.dot(h, wh_ref[...], precision=lax.Precision.DEFAULT,
                 preferred_element_type=F32) + bh_ref[...]
    r = jax.nn.sigmoid(gi[:, :D] + gh[:, :D])
    z = jax.nn.sigmoid(gi[:, D:2 * D] + gh[:, D:2 * D])
    n = jnp.tanh(gi[:, 2 * D:] + r * gh[:, 2 * D:])
    hn = (1.0 - z) * n + z * h
    hn_ref[...] = hn
    hs_ref[...] = jnp.dot(hn, wsrc_ref[...], precision=HIGH,
                          preferred_element_type=F32)
    hd_ref[...] = jnp.dot(hn, wdst_ref[...], precision=HIGH,
                          preferred_element_type=F32)


def _tc_gru(partials, h, gru_wi, gru_wh, gru_bi, gru_bh, w_src, w_dst):
    grid = N // NBLK
    return pl.pallas_call(
        _gru_body,
        grid=(grid,),
        in_specs=[
            pl.BlockSpec((1, NBLK, D), lambda i: (0, i, 0)),
            pl.BlockSpec((1, NBLK, D), lambda i: (1, i, 0)),
            pl.BlockSpec((NBLK, D), lambda i: (i, 0)),
            pl.BlockSpec((D, 3 * D), lambda i: (0, 0)),
            pl.BlockSpec((D, 3 * D), lambda i: (0, 0)),
            pl.BlockSpec((1, 3 * D), lambda i: (0, 0)),
            pl.BlockSpec((1, 3 * D), lambda i: (0, 0)),
            pl.BlockSpec((D, D), lambda i: (0, 0)),
            pl.BlockSpec((D, D), lambda i: (0, 0)),
        ],
        out_specs=[
            pl.BlockSpec((NBLK, D), lambda i: (i, 0)),
            pl.BlockSpec((NBLK, D), lambda i: (i, 0)),
            pl.BlockSpec((NBLK, D), lambda i: (i, 0)),
        ],
        out_shape=[jax.ShapeDtypeStruct((N, D), F32)] * 3,
    )(partials, partials, h, gru_wi, gru_wh, gru_bi, gru_bh, w_src, w_dst)


def _s2s_body(h_ref, nidx_ref, lwi_ref, lwh_ref, lb_ref,
              dw0_ref, dw1_ref, dw2_ref, db_ref,
              tp_ref, t1_ref, t2_ref,
              qstar, hs_s, cs_s, num_s, den_s, mx_s):
    qstar[...] = jnp.zeros((G, 2 * D), F32)
    hs_s[...] = jnp.zeros((G, D), F32)
    cs_s[...] = jnp.zeros((G, D), F32)
    giota = lax.broadcasted_iota(jnp.int32, (1, G), 1)
    nblocks = N // NBLK

    def step(_, carry):
        gates = (jnp.dot(qstar[...], lwi_ref[...], precision=HIGH,
                         preferred_element_type=F32)
                 + jnp.dot(hs_s[...], lwh_ref[...], precision=HIGH,
                           preferred_element_type=F32)
                 + lb_ref[...])
        i_g = jax.nn.sigmoid(gates[:, :D])
        f_g = jax.nn.sigmoid(gates[:, D:2 * D])
        g_g = jnp.tanh(gates[:, 2 * D:3 * D])
        o_g = jax.nn.sigmoid(gates[:, 3 * D:])
        cs = f_g * cs_s[...] + i_g * g_g
        hs = o_g * jnp.tanh(cs)
        cs_s[...] = cs
        hs_s[...] = hs

        mx_s[...] = jnp.full((1, G), -3e38, F32)

        def blk_max(b, c2):
            hb = h_ref[pl.ds(b * NBLK, NBLK), :]
            nb = nidx_ref[pl.ds(b * NBLK, NBLK), :]
            e_all = lax.dot_general(hb, hs, (((1,), (1,)), ((), ())),
                                    precision=HIGH,
                                    preferred_element_type=F32)
            onehot = nb == giota
            em = jnp.where(onehot, e_all, -3e38)
            mx_s[...] = jnp.maximum(mx_s[...], jnp.max(em, axis=0,
                                                       keepdims=True))
            return c2

        lax.fori_loop(0, nblocks, blk_max, 0, unroll=False)

        num_s[...] = jnp.zeros((G, D), F32)
        den_s[...] = jnp.zeros((G, 8), F32)
        mx = mx_s[...]

        def blk_sum(b, c2):
            hb = h_ref[pl.ds(b * NBLK, NBLK), :]
            nb = nidx_ref[pl.ds(b * NBLK, NBLK), :]
            e_all = lax.dot_general(hb, hs, (((1,), (1,)), ((), ())),
                                    precision=HIGH,
                                    preferred_element_type=F32)
            onehot = nb == giota
            a = jnp.where(onehot, jnp.exp(e_all - mx), 0.0)
            num_s[...] += lax.dot_general(a, hb, (((0,), (0,)), ((), ())),
                                          precision=HIGH,
                                          preferred_element_type=F32)
            ones_col = jnp.ones((NBLK, 8), F32)
            den_s[...] += lax.dot_general(a, ones_col, (((0,), (0,)), ((), ())),
                                          precision=HIGH,
                                          preferred_element_type=F32)
            return c2

        lax.fori_loop(0, nblocks, blk_sum, 0, unroll=False)
        den = den_s[:, :1]
        rvec = num_s[...] / jnp.where(den > 0.0, den, 1.0)
        qstar[:, :D] = hs
        qstar[:, D:] = rvec
        return carry

    lax.fori_loop(0, S2S_STEPS, step, 0, unroll=False)

    pool = qstar[...]
    h_g = h_ref[0:G, :]
    tp_ref[...] = jnp.dot(pool, dw0_ref[...], precision=HIGH,
                          preferred_element_type=F32) + db_ref[...]
    t1_ref[...] = jnp.dot(h_g, dw1_ref[...], precision=HIGH,
                          preferred_element_type=F32)
    t2_ref[...] = jnp.dot(h_g, dw2_ref[...], precision=HIGH,
                          preferred_element_type=F32)


def _tc_s2s(h, nidx2d, lstm_wi, lstm_wh, lstm_b, dw0, dw1, dw2, db):
    return pl.pallas_call(
        _s2s_body,
        in_specs=[pl.BlockSpec(memory_space=pltpu.VMEM)] * 9,
        out_specs=[pl.BlockSpec(memory_space=pltpu.VMEM)] * 3,
        out_shape=[jax.ShapeDtypeStruct((G, HID), F32)] * 3,
        scratch_shapes=[
            pltpu.VMEM((G, 2 * D), F32),
            pltpu.VMEM((G, D), F32),
            pltpu.VMEM((G, D), F32),
            pltpu.VMEM((G, D), F32),
            pltpu.VMEM((G, 8), F32),
            pltpu.VMEM((1, G), F32),
        ],
    )(h, nidx2d, lstm_wi, lstm_wh, lstm_b, dw0, dw1, dw2, db)


def _lane_gather(v, idx):
    dnums = lax.GatherDimensionNumbers(offset_dims=(),
                                       collapsed_slice_dims=(0,),
                                       start_index_map=(0,))
    return lax.gather(v, idx[:, None], dnums, (1,),
                      mode=lax.GatherScatterMode.PROMISE_IN_BOUNDS)


def _sc_couple_body(tp_hbm, t1_hbm, t2_hbm, b_hbm, a0_hbm, a1_hbm,
                    w_hbm, pb_hbm, out_hbm,
                    bv0, v00, v10, bv1, v01, v11,
                    buf_p0, buf_00, buf_10, buf_p1, buf_01, buf_11,
                    w_v, pb_v, outall,
                    sem_i0, sem_i1,
                    sem_p0, sem_00, sem_10, sem_p1, sem_01, sem_11,
                    sem_o):
    cid = lax.axis_index("c")
    sid = lax.axis_index("s")
    wid = sid * NC + cid
    lane = lax.broadcasted_iota(jnp.int32, (LANES,), 0)

    pltpu.sync_copy(w_hbm, w_v)
    pltpu.sync_copy(pb_hbm, pb_v)
    pb = pb_v[...]

    isets = ((bv0, v00, v10, sem_i0), (bv1, v01, v11, sem_i1))
    bsets = ((buf_p0, buf_00, buf_10, sem_p0, sem_00, sem_10),
             (buf_p1, buf_01, buf_11, sem_p1, sem_01, sem_11))

    def valid(k):
        return wid + k * NW < N_CCHUNK

    def prefetch_idx(s, k):
        bv, v0, v1, sem_i = isets[s]
        base = (wid + k * NW) * CK
        pltpu.async_copy(b_hbm.at[pl.ds(base, CK)], bv, sem_i)
        pltpu.async_copy(a0_hbm.at[pl.ds(base, CK)], v0, sem_i)
        pltpu.async_copy(a1_hbm.at[pl.ds(base, CK)], v1, sem_i)

    def wait_idx(s):
        bv, v0, v1, sem_i = isets[s]
        pltpu.make_async_copy(b_hbm.at[pl.ds(0, CK)], bv, sem_i).wait()
        pltpu.make_async_copy(a0_hbm.at[pl.ds(0, CK)], v0, sem_i).wait()
        pltpu.make_async_copy(a1_hbm.at[pl.ds(0, CK)], v1, sem_i).wait()

    def issue(s, k):
        bv, v0, v1, _ = isets[s]
        buf_p, buf_0, buf_1, sem_p, sem_0, sem_1 = bsets[s]
        pltpu.async_copy(tp_hbm.at[bv], buf_p, sem_p)
        pltpu.async_copy(t1_hbm.at[v0], buf_0, sem_0)
        pltpu.async_copy(t2_hbm.at[v1], buf_1, sem_1)

    def drain(s):
        bv, v0, v1, _ = isets[s]
        buf_p, buf_0, buf_1, sem_p, sem_0, sem_1 = bsets[s]
        pltpu.make_async_copy(tp_hbm.at[bv], buf_p, sem_p).wait()
        pltpu.make_async_copy(t1_hbm.at[v0], buf_0, sem_0).wait()
        pltpu.make_async_copy(t2_hbm.at[v1], buf_1, sem_1).wait()

    def out_slice(k):
        return out_hbm.at[pl.ds((wid + k * NW) * CK, CK)]

    def compute(s, k):
        buf_p, buf_0, buf_1, _, _, _ = bsets[s]

        def row(j, res):
            def col(kk, acc2):
                sl = pl.ds(kk * LANES, LANES)
                v = buf_p[j, sl] + buf_0[j, sl] + buf_1[j, sl]
                v = jnp.maximum(v, 0.0)
                return acc2 + v * w_v[sl]

            acc2 = lax.fori_loop(0, HID // LANES, col,
                                 jnp.zeros((LANES,), F32), unroll=8)
            for kk in (1, 2, 4, 8):
                acc2 = acc2 + _lane_gather(acc2, lane ^ kk)
            return jnp.where(lane == j, acc2, res)

        res = lax.fori_loop(0, CK, row, jnp.zeros((LANES,), F32),
                            unroll=False)
        val = (YR1 - YR0) / (1.0 + jnp.exp(-(res + pb))) + YR0
        outall[k] = val
        pltpu.async_copy(outall.at[k], out_slice(k), sem_o)

    def stage(s, k):
        s2 = 1 - s

        @pl.when(valid(k + 1))
        def _():
            wait_idx(s2)
            issue(s2, k + 1)

        @pl.when(valid(k))
        def _():
            drain(s)
            compute(s, k)

        @pl.when(valid(k + 2))
        def _():
            prefetch_idx(s, k + 2)

    prefetch_idx(0, 0)
    wait_idx(0)
    issue(0, 0)

    @pl.when(valid(1))
    def _():
        prefetch_idx(1, 1)

    def pair_body(p, carry):
        stage(0, 2 * p)
        stage(1, 2 * p + 1)
        return carry

    lax.fori_loop(0, CPAIRS, pair_body, 0, unroll=False)

    def drain_out(k, carry):
        @pl.when(valid(k))
        def _():
            pltpu.make_async_copy(outall.at[k], out_slice(k), sem_o).wait()

        return carry

    lax.fori_loop(0, CKIDX, drain_out, 0, unroll=False)


def _sc_couple(tp, t1, t2, bidx, a0, a1, w, pb):
    mesh = plsc.VectorSubcoreMesh(core_axis_name="c", subcore_axis_name="s")
    fn = pl.kernel(
        _sc_couple_body,
        out_type=jax.ShapeDtypeStruct((C,), F32),
        mesh=mesh,
        scratch_types=[
            pltpu.VMEM((CK,), jnp.int32),
            pltpu.VMEM((CK,), jnp.int32),
            pltpu.VMEM((CK,), jnp.int32),
            pltpu.VMEM((CK,), jnp.int32),
            pltpu.VMEM((CK,), jnp.int32),
            pltpu.VMEM((CK,), jnp.int32),
            pltpu.VMEM((CK, HID), F32),
            pltpu.VMEM((CK, HID), F32),
            pltpu.VMEM((CK, HID), F32),
            pltpu.VMEM((CK, HID), F32),
            pltpu.VMEM((CK, HID), F32),
            pltpu.VMEM((CK, HID), F32),
            pltpu.VMEM((HID,), F32),
            pltpu.VMEM((LANES,), F32),
            pltpu.VMEM((CKIDX, CK), F32),
            pltpu.SemaphoreType.DMA,
            pltpu.SemaphoreType.DMA,
            pltpu.SemaphoreType.DMA,
            pltpu.SemaphoreType.DMA,
            pltpu.SemaphoreType.DMA,
            pltpu.SemaphoreType.DMA,
            pltpu.SemaphoreType.DMA,
            pltpu.SemaphoreType.DMA,
            pltpu.SemaphoreType.DMA,
        ],
    )
    return fn(tp, t1, t2, bidx, a0, a1, w, pb)


def kernel(node, edge, edge_index, node_index, coupling_index, bond_type,
           x_atomic, W_pre, b_pre, msg_W, msg_b, gru_Wi, gru_Wh, gru_bi,
           gru_bh, lstm_Wi, lstm_Wh, lstm_b, dense_W, dense_b, pred_W,
           pred_b):
    w_src = msg_W[:D]
    w_dst = msg_W[D:2 * D]
    w_e = msg_W[2 * D:]
    b_pre2 = b_pre.reshape(1, D)
    msg_b2 = msg_b.reshape(1, D)
    gru_bi2 = gru_bi.reshape(1, 3 * D)
    gru_bh2 = gru_bh.reshape(1, 3 * D)
    lstm_b2 = lstm_b.reshape(1, 4 * D)
    dense_b2 = dense_b.reshape(1, HID)
    src = edge_index[0]
    dst = edge_index[1]
    nidx2d = node_index.reshape(N, 1)
    bidx = coupling_index[:, 3]
    a0 = coupling_index[:, 0]
    a1 = coupling_index[:, 1]
    zeros_tile = jnp.zeros((TILE_ROWS, D), F32)
    pred_w1 = pred_W.reshape(HID)
    pred_b16 = jnp.full((LANES,), pred_b[0], F32)

    h, hs, hd = _tc_pre(node, W_pre, b_pre2, w_src, w_dst)
    eb = _tc_eproj(edge, w_e, msg_b2)

    for _ in range(T_STEPS):
        partials = _sc_edge(hs, hd, eb, edge_index, zeros_tile)
        h, hs, hd = _tc_gru(partials, h, gru_Wi, gru_Wh,
                            gru_bi2, gru_bh2, w_src, w_dst)

    tp, t1, t2 = _tc_s2s(h, nidx2d, lstm_Wi, lstm_Wh, lstm_b2,
                         dense_W[:2 * D], dense_W[2 * D:3 * D],
                         dense_W[3 * D:], dense_b2)

    out = _sc_couple(tp, t1, t2, bidx, a0, a1, pred_w1, pred_b16)
    return out.reshape(C, 1)


# speedup vs baseline: 1.5168x; 1.5168x over previous
import functools

import jax
import jax.numpy as jnp
from jax import lax
from jax.experimental import pallas as pl
from jax.experimental.pallas import tpu as pltpu
from jax.experimental.pallas import tpu_sc as plsc

N = 10000
E = 320000
D = 128
ED = 16
G = 512
C = 20000
T_STEPS = 3
S2S_STEPS = 3
HID = 1024
YR0, YR1 = -36.2186, 204.88

F32 = jnp.float32
HIGH = lax.Precision.HIGHEST

NC = 2
NS = 16
NW = NC * NS
LANES = 16

TILE_ROWS = 632
LAST_ROWS = N - (NS - 1) * TILE_ROWS

EK = 128
N_ECHUNK = E // EK
ECHUNK_ITERS = -(-N_ECHUNK // NW)
EPAIRS = (ECHUNK_ITERS + 1) // 2

CK = 16
N_CCHUNK = C // CK
CKIDX = 48
CPAIRS = CKIDX // 2

NBLK = 1000
EBLK = 4000


def _pre_body(node_ref, wpre_ref, bpre_ref, wsrc_ref, wdst_ref,
              h_ref, hs_ref, hd_ref):
    h = jnp.maximum(
        jnp.dot(node_ref[...], wpre_ref[...], precision=lax.Precision.DEFAULT,
                preferred_element_type=F32) + bpre_ref[...], 0.0)
    h_ref[...] = h
    hs_ref[...] = jnp.dot(h, wsrc_ref[...], precision=HIGH,
                          preferred_element_type=F32)
    hd_ref[...] = jnp.dot(h, wdst_ref[...], precision=HIGH,
                          preferred_element_type=F32)


def _tc_pre(node, w_pre, b_pre, w_src, w_dst):
    grid = N // NBLK
    return pl.pallas_call(
        _pre_body,
        grid=(grid,),
        in_specs=[
            pl.BlockSpec((NBLK, D), lambda i: (i, 0)),
            pl.BlockSpec((D, D), lambda i: (0, 0)),
            pl.BlockSpec((1, D), lambda i: (0, 0)),
            pl.BlockSpec((D, D), lambda i: (0, 0)),
            pl.BlockSpec((D, D), lambda i: (0, 0)),
        ],
        out_specs=[
            pl.BlockSpec((NBLK, D), lambda i: (i, 0)),
            pl.BlockSpec((NBLK, D), lambda i: (i, 0)),
            pl.BlockSpec((NBLK, D), lambda i: (i, 0)),
        ],
        out_shape=[jax.ShapeDtypeStruct((N, D), F32)] * 3,
    )(node, w_pre, b_pre, w_src, w_dst)


def _eproj_body(edge_ref, we_ref, mb_ref, out_ref):
    out_ref[...] = jnp.dot(edge_ref[...], we_ref[...], precision=HIGH,
                           preferred_element_type=F32) + mb_ref[...]


def _tc_eproj(edge, w_e, msg_b):
    grid = E // EBLK
    return pl.pallas_call(
        _eproj_body,
        grid=(grid,),
        in_specs=[
            pl.BlockSpec((EBLK, ED), lambda i: (i, 0)),
            pl.BlockSpec((ED, D), lambda i: (0, 0)),
            pl.BlockSpec((1, D), lambda i: (0, 0)),
        ],
        out_specs=pl.BlockSpec((EBLK, D), lambda i: (i, 0)),
        out_shape=jax.ShapeDtypeStruct((E, D), F32),
    )(edge, w_e, msg_b)


def _sc_edge_body(hs_hbm, hd_hbm, eb_hbm, ei_hbm, zeros_hbm,
                  out_hbm,
                  sdv, buf_a, buf_b, buf_c, acc, sem_a, sem_b):
    cid = lax.axis_index("c")
    sid = lax.axis_index("s")
    wid = sid * NC + cid
    row_base = sid * TILE_ROWS

    @pl.when(sid < NS - 1)
    def _():
        pltpu.sync_copy(zeros_hbm, acc.at[pl.ds(row_base, TILE_ROWS)])

    @pl.when(sid == NS - 1)
    def _():
        pltpu.sync_copy(zeros_hbm.at[pl.ds(0, LAST_ROWS)],
                        acc.at[pl.ds(row_base, LAST_ROWS)])

    plsc.subcore_barrier()

    def chunk_body(k, carry):
        chunk = wid + k * NW

        @pl.when(chunk < N_ECHUNK)
        def _():
            base = chunk * EK
            pltpu.sync_copy(ei_hbm.at[:, pl.ds(base, EK)], sdv)
            cp_a = pltpu.async_copy(hs_hbm.at[sdv.at[0]], buf_a, sem_a)
            cp_b = pltpu.async_copy(hd_hbm.at[sdv.at[1]], buf_b, sem_b)
            pltpu.sync_copy(eb_hbm.at[pl.ds(base, EK)], buf_c)
            cp_a.wait()
            cp_b.wait()

            def row_body(i, c2):
                for col in range(D // LANES):
                    sl = pl.ds(col * LANES, LANES)
                    v = buf_a[i, sl] + buf_b[i, sl] + buf_c[i, sl]
                    buf_a[i, sl] = jnp.maximum(v, 0.0)
                return c2

            lax.fori_loop(0, EK, row_body, 0, unroll=False)
            pltpu.sync_copy(buf_a, acc.at[sdv.at[1]], add=True)

        return carry

    lax.fori_loop(0, ECHUNK_ITERS, chunk_body, 0, unroll=False)
    plsc.subcore_barrier()

    @pl.when(sid < NS - 1)
    def _():
        pltpu.sync_copy(acc.at[pl.ds(row_base, TILE_ROWS)],
                        out_hbm.at[cid, pl.ds(row_base, TILE_ROWS)])

    @pl.when(sid == NS - 1)
    def _():
        pltpu.sync_copy(acc.at[pl.ds(row_base, LAST_ROWS)],
                        out_hbm.at[cid, pl.ds(row_base, LAST_ROWS)])


def _sc_edge(hs, hd, eb, edge_index, zeros_tile):
    mesh = plsc.VectorSubcoreMesh(core_axis_name="c", subcore_axis_name="s")
    fn = pl.kernel(
        _sc_edge_body,
        out_type=jax.ShapeDtypeStruct((NC, N, D), F32),
        mesh=mesh,
        scratch_types=[
            pltpu.VMEM((2, EK), jnp.int32),
            pltpu.VMEM((EK, D), F32),
            pltpu.VMEM((EK, D), F32),
            pltpu.VMEM((EK, D), F32),
            pltpu.VMEM_SHARED((N, D), F32),
            pltpu.SemaphoreType.DMA,
            pltpu.SemaphoreType.DMA,
        ],
    )
    return fn(hs, hd, eb, edge_index, zeros_tile)


def _gru_body(p0_ref, p1_ref, h_ref, wi_ref, wh_ref, bi_ref, bh_ref,
              wsrc_ref, wdst_ref, hn_ref, hs_ref, hd_ref):
    m = p0_ref[0] + p1_ref[0]
    h = h_ref[...]
    gi = jnp.dot(m, wi_ref[...], precision=lax.Precision.DEFAULT,
                 preferred_element_type=F32) + bi_ref[...]
    gh = jnp.dot(h, wh_ref[...], precision=lax.Precision.DEFAULT,
                 preferred_element_type=F32) + bh_ref[...]
    r = jax.nn.sigmoid(gi[:, :D] + gh[:, :D])
    z = jax.nn.sigmoid(gi[:, D:2 * D] + gh[:, D:2 * D])
    n = jnp.tanh(gi[:, 2 * D:] + r * gh[:, 2 * D:])
    hn = (1.0 - z) * n + z * h
    hn_ref[...] = hn
    hs_ref[...] = jnp.dot(hn, wsrc_ref[...], precision=HIGH,
                          preferred_element_type=F32)
    hd_ref[...] = jnp.dot(hn, wdst_ref[...], precision=HIGH,
                          preferred_element_type=F32)


def _tc_gru(partials, h, gru_wi, gru_wh, gru_bi, gru_bh, w_src, w_dst):
    grid = N // NBLK
    return pl.pallas_call(
        _gru_body,
        grid=(grid,),
        in_specs=[
            pl.BlockSpec((1, NBLK, D), lambda i: (0, i, 0)),
            pl.BlockSpec((1, NBLK, D), lambda i: (1, i, 0)),
            pl.BlockSpec((NBLK, D), lambda i: (i, 0)),
            pl.BlockSpec((D, 3 * D), lambda i: (0, 0)),
            pl.BlockSpec((D, 3 * D), lambda i: (0, 0)),
            pl.BlockSpec((1, 3 * D), lambda i: (0, 0)),
            pl.BlockSpec((1, 3 * D), lambda i: (0, 0)),
            pl.BlockSpec((D, D), lambda i: (0, 0)),
            pl.BlockSpec((D, D), lambda i: (0, 0)),
        ],
        out_specs=[
            pl.BlockSpec((NBLK, D), lambda i: (i, 0)),
            pl.BlockSpec((NBLK, D), lambda i: (i, 0)),
            pl.BlockSpec((NBLK, D), lambda i: (i, 0)),
        ],
        out_shape=[jax.ShapeDtypeStruct((N, D), F32)] * 3,
    )(partials, partials, h, gru_wi, gru_wh, gru_bi, gru_bh, w_src, w_dst)


def _s2s_body(h_ref, nidx_ref, lwi_ref, lwh_ref, lb_ref,
              dw0_ref, dw1_ref, dw2_ref, db_ref,
              tp_ref, t1_ref, t2_ref,
              qstar, hs_s, cs_s, num_s, den_s, mx_s):
    qstar[...] = jnp.zeros((G, 2 * D), F32)
    hs_s[...] = jnp.zeros((G, D), F32)
    cs_s[...] = jnp.zeros((G, D), F32)
    giota = lax.broadcasted_iota(jnp.int32, (1, G), 1)
    nblocks = N // NBLK

    def step(_, carry):
        gates = (jnp.dot(qstar[...], lwi_ref[...], precision=HIGH,
                         preferred_element_type=F32)
                 + jnp.dot(hs_s[...], lwh_ref[...], precision=HIGH,
                           preferred_element_type=F32)
                 + lb_ref[...])
        i_g = jax.nn.sigmoid(gates[:, :D])
        f_g = jax.nn.sigmoid(gates[:, D:2 * D])
        g_g = jnp.tanh(gates[:, 2 * D:3 * D])
        o_g = jax.nn.sigmoid(gates[:, 3 * D:])
        cs = f_g * cs_s[...] + i_g * g_g
        hs = o_g * jnp.tanh(cs)
        cs_s[...] = cs
        hs_s[...] = hs

        mx_s[...] = jnp.full((1, G), -3e38, F32)

        def blk_max(b, c2):
            hb = h_ref[pl.ds(b * NBLK, NBLK), :]
            nb = nidx_ref[pl.ds(b * NBLK, NBLK), :]
            e_all = lax.dot_general(hb, hs, (((1,), (1,)), ((), ())),
                                    precision=HIGH,
                                    preferred_element_type=F32)
            onehot = nb == giota
            em = jnp.where(onehot, e_all, -3e38)
            mx_s[...] = jnp.maximum(mx_s[...], jnp.max(em, axis=0,
                                                       keepdims=True))
            return c2

        lax.fori_loop(0, nblocks, blk_max, 0, unroll=False)

        num_s[...] = jnp.zeros((G, D), F32)
        den_s[...] = jnp.zeros((G, 8), F32)
        mx = mx_s[...]

        def blk_sum(b, c2):
            hb = h_ref[pl.ds(b * NBLK, NBLK), :]
            nb = nidx_ref[pl.ds(b * NBLK, NBLK), :]
            e_all = lax.dot_general(hb, hs, (((1,), (1,)), ((), ())),
                                    precision=HIGH,
                                    preferred_element_type=F32)
            onehot = nb == giota
            a = jnp.where(onehot, jnp.exp(e_all - mx), 0.0)
            num_s[...] += lax.dot_general(a, hb, (((0,), (0,)), ((), ())),
                                          precision=HIGH,
                                          preferred_element_type=F32)
            ones_col = jnp.ones((NBLK, 8), F32)
            den_s[...] += lax.dot_general(a, ones_col, (((0,), (0,)), ((), ())),
                                          precision=HIGH,
                                          preferred_element_type=F32)
            return c2

        lax.fori_loop(0, nblocks, blk_sum, 0, unroll=False)
        den = den_s[:, :1]
        rvec = num_s[...] / jnp.where(den > 0.0, den, 1.0)
        qstar[:, :D] = hs
        qstar[:, D:] = rvec
        return carry

    lax.fori_loop(0, S2S_STEPS, step, 0, unroll=False)

    pool = qstar[...]
    h_g = h_ref[0:G, :]
    tp_ref[...] = jnp.dot(pool, dw0_ref[...], precision=HIGH,
                          preferred_element_type=F32) + db_ref[...]
    t1_ref[...] = jnp.dot(h_g, dw1_ref[...], precision=HIGH,
                          preferred_element_type=F32)
    t2_ref[...] = jnp.dot(h_g, dw2_ref[...], precision=HIGH,
                          preferred_element_type=F32)


def _tc_s2s(h, nidx2d, lstm_wi, lstm_wh, lstm_b, dw0, dw1, dw2, db):
    return pl.pallas_call(
        _s2s_body,
        in_specs=[pl.BlockSpec(memory_space=pltpu.VMEM)] * 9,
        out_specs=[pl.BlockSpec(memory_space=pltpu.VMEM)] * 3,
        out_shape=[jax.ShapeDtypeStruct((G, HID), F32)] * 3,
        scratch_shapes=[
            pltpu.VMEM((G, 2 * D), F32),
            pltpu.VMEM((G, D), F32),
            pltpu.VMEM((G, D), F32),
            pltpu.VMEM((G, D), F32),
            pltpu.VMEM((G, 8), F32),
            pltpu.VMEM((1, G), F32),
        ],
    )(h, nidx2d, lstm_wi, lstm_wh, lstm_b, dw0, dw1, dw2, db)


def _lane_gather(v, idx):
    dnums = lax.GatherDimensionNumbers(offset_dims=(),
                                       collapsed_slice_dims=(0,),
                                       start_index_map=(0,))
    return lax.gather(v, idx[:, None], dnums, (1,),
                      mode=lax.GatherScatterMode.PROMISE_IN_BOUNDS)


def _sc_couple_body(tp_hbm, t1_hbm, t2_hbm, b_hbm, a0_hbm, a1_hbm,
                    w_hbm, pb_hbm, out_hbm,
                    bv0, v00, v10, bv1, v01, v11,
                    buf_p0, buf_00, buf_10, buf_p1, buf_01, buf_11,
                    w_v, pb_v, outall,
                    sem_i0, sem_i1,
                    sem_p0, sem_00, sem_10, sem_p1, sem_01, sem_11,
                    sem_o):
    cid = lax.axis_index("c")
    sid = lax.axis_index("s")
    wid = sid * NC + cid
    lane = lax.broadcasted_iota(jnp.int32, (LANES,), 0)

    pltpu.sync_copy(w_hbm, w_v)
    pltpu.sync_copy(pb_hbm, pb_v)
    pb = pb_v[...]

    isets = ((bv0, v00, v10, sem_i0), (bv1, v01, v11, sem_i1))
    bsets = ((buf_p0, buf_00, buf_10, sem_p0, sem_00, sem_10),
             (buf_p1, buf_01, buf_11, sem_p1, sem_01, sem_11))

    def valid(k):
        return wid + k * NW < N_CCHUNK

    def prefetch_idx(s, k):
        bv, v0, v1, sem_i = isets[s]
        base = (wid + k * NW) * CK
        pltpu.async_copy(b_hbm.at[pl.ds(base, CK)], bv, sem_i)
        pltpu.async_copy(a0_hbm.at[pl.ds(base, CK)], v0, sem_i)
        pltpu.async_copy(a1_hbm.at[pl.ds(base, CK)], v1, sem_i)

    def wait_idx(s):
        bv, v0, v1, sem_i = isets[s]
        pltpu.make_async_copy(b_hbm.at[pl.ds(0, CK)], bv, sem_i).wait()
        pltpu.make_async_copy(a0_hbm.at[pl.ds(0, CK)], v0, sem_i).wait()
        pltpu.make_async_copy(a1_hbm.at[pl.ds(0, CK)], v1, sem_i).wait()

    def issue(s, k):
        bv, v0, v1, _ = isets[s]
        buf_p, buf_0, buf_1, sem_p, sem_0, sem_1 = bsets[s]
        pltpu.async_copy(tp_hbm.at[bv], buf_p, sem_p)
        pltpu.async_copy(t1_hbm.at[v0], buf_0, sem_0)
        pltpu.async_copy(t2_hbm.at[v1], buf_1, sem_1)

    def drain(s):
        bv, v0, v1, _ = isets[s]
        buf_p, buf_0, buf_1, sem_p, sem_0, sem_1 = bsets[s]
        pltpu.make_async_copy(tp_hbm.at[bv], buf_p, sem_p).wait()
        pltpu.make_async_copy(t1_hbm.at[v0], buf_0, sem_0).wait()
        pltpu.make_async_copy(t2_hbm.at[v1], buf_1, sem_1).wait()

    def out_slice(k):
        return out_hbm.at[pl.ds((wid + k * NW) * CK, CK)]

    def compute(s, k):
        buf_p, buf_0, buf_1, _, _, _ = bsets[s]

        def row(j, res):
            def col(kk, acc2):
                sl = pl.ds(kk * LANES, LANES)
                v = buf_p[j, sl] + buf_0[j, sl] + buf_1[j, sl]
                v = jnp.maximum(v, 0.0)
                return acc2 + v * w_v[sl]

            acc2 = lax.fori_loop(0, HID // LANES, col,
                                 jnp.zeros((LANES,), F32), unroll=8)
            for kk in (1, 2, 4, 8):
                acc2 = acc2 + _lane_gather(acc2, lane ^ kk)
            return jnp.where(lane == j, acc2, res)

        res = lax.fori_loop(0, CK, row, jnp.zeros((LANES,), F32),
                            unroll=False)
        val = (YR1 - YR0) / (1.0 + jnp.exp(-(res + pb))) + YR0
        outall[k] = val
        pltpu.async_copy(outall.at[k], out_slice(k), sem_o)

    def stage(s, k):
        s2 = 1 - s

        @pl.when(valid(k + 1))
        def _():
            wait_idx(s2)
            issue(s2, k + 1)

        @pl.when(valid(k))
        def _():
            drain(s)
            compute(s, k)

        @pl.when(valid(k + 2))
        def _():
            prefetch_idx(s, k + 2)

    prefetch_idx(0, 0)
    wait_idx(0)
    issue(0, 0)

    @pl.when(valid(1))
    def _():
        prefetch_idx(1, 1)

    def pair_body(p, carry):
        stage(0, 2 * p)
        stage(1, 2 * p + 1)
        return carry

    lax.fori_loop(0, CPAIRS, pair_body, 0, unroll=False)

    def drain_out(k, carry):
        @pl.when(valid(k))
        def _():
            pltpu.make_async_copy(outall.at[k], out_slice(k), sem_o).wait()

        return carry

    lax.fori_loop(0, CKIDX, drain_out, 0, unroll=False)


def _sc_couple(tp, t1, t2, bidx, a0, a1, w, pb):
    mesh = plsc.VectorSubcoreMesh(core_axis_name="c", subcore_axis_name="s")
    fn = pl.kernel(
        _sc_couple_body,
        out_type=jax.ShapeDtypeStruct((C,), F32),
        mesh=mesh,
        scratch_types=[
            pltpu.VMEM((CK,), jnp.int32),
            pltpu.VMEM((CK,), jnp.int32),
            pltpu.VMEM((CK,), jnp.int32),
            pltpu.VMEM((CK,), jnp.int32),
            pltpu.VMEM((CK,), jnp.int32),
            pltpu.VMEM((CK,), jnp.int32),
            pltpu.VMEM((CK, HID), F32),
            pltpu.VMEM((CK, HID), F32),
            pltpu.VMEM((CK, HID), F32),
            pltpu.VMEM((CK, HID), F32),
            pltpu.VMEM((CK, HID), F32),
            pltpu.VMEM((CK, HID), F32),
            pltpu.VMEM((HID,), F32),
            pltpu.VMEM((LANES,), F32),
            pltpu.VMEM((CKIDX, CK), F32),
            pltpu.SemaphoreType.DMA,
            pltpu.SemaphoreType.DMA,
            pltpu.SemaphoreType.DMA,
            pltpu.SemaphoreType.DMA,
            pltpu.SemaphoreType.DMA,
            pltpu.SemaphoreType.DMA,
            pltpu.SemaphoreType.DMA,
            pltpu.SemaphoreType.DMA,
            pltpu.SemaphoreType.DMA,
        ],
    )
    return fn(tp, t1, t2, bidx, a0, a1, w, pb)


def kernel(node, edge, edge_index, node_index, coupling_index, bond_type,
           x_atomic, W_pre, b_pre, msg_W, msg_b, gru_Wi, gru_Wh, gru_bi,
           gru_bh, lstm_Wi, lstm_Wh, lstm_b, dense_W, dense_b, pred_W,
           pred_b):
    w_src = msg_W[:D]
    w_dst = msg_W[D:2 * D]
    w_e = msg_W[2 * D:]
    b_pre2 = b_pre.reshape(1, D)
    msg_b2 = msg_b.reshape(1, D)
    gru_bi2 = gru_bi.reshape(1, 3 * D)
    gru_bh2 = gru_bh.reshape(1, 3 * D)
    lstm_b2 = lstm_b.reshape(1, 4 * D)
    dense_b2 = dense_b.reshape(1, HID)
    src = edge_index[0]
    dst = edge_index[1]
    nidx2d = node_index.reshape(N, 1)
    bidx = coupling_index[:, 3]
    a0 = coupling_index[:, 0]
    a1 = coupling_index[:, 1]
    zeros_tile = jnp.zeros((TILE_ROWS, D), F32)
    pred_w1 = pred_W.reshape(HID)
    pred_b16 = jnp.full((LANES,), pred_b[0], F32)

    h, hs, hd = _tc_pre(node, W_pre, b_pre2, w_src, w_dst)
    eb = _tc_eproj(edge, w_e, msg_b2)

    for _ in range(T_STEPS):
        partials = _sc_edge(hs, hd, eb, edge_index, zeros_tile)
        h, hs, hd = _tc_gru(partials, h, gru_Wi, gru_Wh,
                            gru_bi2, gru_bh2, w_src, w_dst)

    tp, t1, t2 = _tc_s2s(h, nidx2d, lstm_Wi, lstm_Wh, lstm_b2,
                         dense_W[:2 * D], dense_W[2 * D:3 * D],
                         dense_W[3 * D:], dense_b2)

    out = _sc_couple(tp, t1, t2, bidx, a0, a1, pred_w1, pred_b16)
    return out.reshape(C, 1)


# speedup vs baseline: 1.5855x; 1.0453x over previous
import functools

import jax
import jax.numpy as jnp
from jax import lax
from jax.experimental import pallas as pl
from jax.experimental.pallas import tpu as pltpu
from jax.experimental.pallas import tpu_sc as plsc

N = 10000
E = 320000
D = 128
ED = 16
G = 512
C = 20000
T_STEPS = 3
S2S_STEPS = 3
HID = 1024
YR0, YR1 = -36.2186, 204.88

F32 = jnp.float32
HIGH = lax.Precision.HIGHEST

NC = 2
NS = 16
NW = NC * NS
LANES = 16

TILE_ROWS = 632
LAST_ROWS = N - (NS - 1) * TILE_ROWS

EK = 128
N_ECHUNK = E // EK
ECHUNK_ITERS = -(-N_ECHUNK // NW)
EPAIRS = (ECHUNK_ITERS + 1) // 2

CK = 16
N_CCHUNK = C // CK
CKIDX = 48
CPAIRS = CKIDX // 2

NBLK = 1000
EBLK = 4000


def _pre_body(node_ref, wpre_ref, bpre_ref, wsrc_ref, wdst_ref,
              h_ref, hs_ref, hd_ref):
    h = jnp.maximum(
        jnp.dot(node_ref[...], wpre_ref[...], precision=lax.Precision.DEFAULT,
                preferred_element_type=F32) + bpre_ref[...], 0.0)
    h_ref[...] = h
    hs_ref[...] = jnp.dot(h, wsrc_ref[...], precision=lax.Precision.DEFAULT,
                          preferred_element_type=F32)
    hd_ref[...] = jnp.dot(h, wdst_ref[...], precision=lax.Precision.DEFAULT,
                          preferred_element_type=F32)


def _tc_pre(node, w_pre, b_pre, w_src, w_dst):
    grid = N // NBLK
    return pl.pallas_call(
        _pre_body,
        grid=(grid,),
        in_specs=[
            pl.BlockSpec((NBLK, D), lambda i: (i, 0)),
            pl.BlockSpec((D, D), lambda i: (0, 0)),
            pl.BlockSpec((1, D), lambda i: (0, 0)),
            pl.BlockSpec((D, D), lambda i: (0, 0)),
            pl.BlockSpec((D, D), lambda i: (0, 0)),
        ],
        out_specs=[
            pl.BlockSpec((NBLK, D), lambda i: (i, 0)),
            pl.BlockSpec((NBLK, D), lambda i: (i, 0)),
            pl.BlockSpec((NBLK, D), lambda i: (i, 0)),
        ],
        out_shape=[jax.ShapeDtypeStruct((N, D), F32)] * 3,
    )(node, w_pre, b_pre, w_src, w_dst)


def _eproj_body(edge_ref, we_ref, mb_ref, out_ref):
    out_ref[...] = jnp.dot(edge_ref[...], we_ref[...],
                           precision=lax.Precision.DEFAULT,
                           preferred_element_type=F32) + mb_ref[...]


def _tc_eproj(edge, w_e, msg_b):
    grid = E // EBLK
    return pl.pallas_call(
        _eproj_body,
        grid=(grid,),
        in_specs=[
            pl.BlockSpec((EBLK, ED), lambda i: (i, 0)),
            pl.BlockSpec((ED, D), lambda i: (0, 0)),
            pl.BlockSpec((1, D), lambda i: (0, 0)),
        ],
        out_specs=pl.BlockSpec((EBLK, D), lambda i: (i, 0)),
        out_shape=jax.ShapeDtypeStruct((E, D), F32),
    )(edge, w_e, msg_b)


def _sc_edge_body(hs_hbm, hd_hbm, eb_hbm, ei_hbm, zeros_hbm,
                  out_hbm,
                  sdv, buf_a, buf_b, buf_c, acc, sem_a, sem_b):
    cid = lax.axis_index("c")
    sid = lax.axis_index("s")
    wid = sid * NC + cid
    row_base = sid * TILE_ROWS

    @pl.when(sid < NS - 1)
    def _():
        pltpu.sync_copy(zeros_hbm, acc.at[pl.ds(row_base, TILE_ROWS)])

    @pl.when(sid == NS - 1)
    def _():
        pltpu.sync_copy(zeros_hbm.at[pl.ds(0, LAST_ROWS)],
                        acc.at[pl.ds(row_base, LAST_ROWS)])

    plsc.subcore_barrier()

    def chunk_body(k, carry):
        chunk = wid + k * NW

        @pl.when(chunk < N_ECHUNK)
        def _():
            base = chunk * EK
            pltpu.sync_copy(ei_hbm.at[:, pl.ds(base, EK)], sdv)
            cp_a = pltpu.async_copy(hs_hbm.at[sdv.at[0]], buf_a, sem_a)
            cp_b = pltpu.async_copy(hd_hbm.at[sdv.at[1]], buf_b, sem_b)
            pltpu.sync_copy(eb_hbm.at[pl.ds(base, EK)], buf_c)
            cp_a.wait()
            cp_b.wait()

            def row_body(i, c2):
                for col in range(D // LANES):
                    sl = pl.ds(col * LANES, LANES)
                    v = buf_a[i, sl] + buf_b[i, sl] + buf_c[i, sl]
                    buf_a[i, sl] = jnp.maximum(v, 0.0)
                return c2

            lax.fori_loop(0, EK, row_body, 0, unroll=False)
            pltpu.sync_copy(buf_a, acc.at[sdv.at[1]], add=True)

        return carry

    lax.fori_loop(0, ECHUNK_ITERS, chunk_body, 0, unroll=False)
    plsc.subcore_barrier()

    @pl.when(sid < NS - 1)
    def _():
        pltpu.sync_copy(acc.at[pl.ds(row_base, TILE_ROWS)],
                        out_hbm.at[cid, pl.ds(row_base, TILE_ROWS)])

    @pl.when(sid == NS - 1)
    def _():
        pltpu.sync_copy(acc.at[pl.ds(row_base, LAST_ROWS)],
                        out_hbm.at[cid, pl.ds(row_base, LAST_ROWS)])


def _sc_edge(hs, hd, eb, edge_index, zeros_tile):
    mesh = plsc.VectorSubcoreMesh(core_axis_name="c", subcore_axis_name="s")
    fn = pl.kernel(
        _sc_edge_body,
        out_type=jax.ShapeDtypeStruct((NC, N, D), F32),
        mesh=mesh,
        scratch_types=[
            pltpu.VMEM((2, EK), jnp.int32),
            pltpu.VMEM((EK, D), F32),
            pltpu.VMEM((EK, D), F32),
            pltpu.VMEM((EK, D), F32),
            pltpu.VMEM_SHARED((N, D), F32),
            pltpu.SemaphoreType.DMA,
            pltpu.SemaphoreType.DMA,
        ],
    )
    return fn(hs, hd, eb, edge_index, zeros_tile)


def _gru_body(p0_ref, p1_ref, h_ref, wi_ref, wh_ref, bi_ref, bh_ref,
              wsrc_ref, wdst_ref, hn_ref, hs_ref, hd_ref):
    m = p0_ref[0] + p1_ref[0]
    h = h_ref[...]
    gi = jnp.dot(m, wi_ref[...], precision=lax.Precision.DEFAULT,
                 preferred_element_type=F32) + bi_ref[...]
    gh = jnp.dot(h, wh_ref[...], precision=lax.Precision.DEFAULT,
                 preferred_element_type=F32) + bh_ref[...]
    r = jax.nn.sigmoid(gi[:, :D] + gh[:, :D])
    z = jax.nn.sigmoid(gi[:, D:2 * D] + gh[:, D:2 * D])
    n = jnp.tanh(gi[:, 2 * D:] + r * gh[:, 2 * D:])
    hn = (1.0 - z) * n + z * h
    hn_ref[...] = hn
    hs_ref[...] = jnp.dot(hn, wsrc_ref[...], precision=lax.Precision.DEFAULT,
                          preferred_element_type=F32)
    hd_ref[...] = jnp.dot(hn, wdst_ref[...], precision=lax.Precision.DEFAULT,
                          preferred_element_type=F32)


def _tc_gru(partials, h, gru_wi, gru_wh, gru_bi, gru_bh, w_src, w_dst):
    grid = N // NBLK
    return pl.pallas_call(
        _gru_body,
        grid=(grid,),
        in_specs=[
            pl.BlockSpec((1, NBLK, D), lambda i: (0, i, 0)),
            pl.BlockSpec((1, NBLK, D), lambda i: (1, i, 0)),
            pl.BlockSpec((NBLK, D), lambda i: (i, 0)),
            pl.BlockSpec((D, 3 * D), lambda i: (0, 0)),
            pl.BlockSpec((D, 3 * D), lambda i: (0, 0)),
            pl.BlockSpec((1, 3 * D), lambda i: (0, 0)),
            pl.BlockSpec((1, 3 * D), lambda i: (0, 0)),
            pl.BlockSpec((D, D), lambda i: (0, 0)),
            pl.BlockSpec((D, D), lambda i: (0, 0)),
        ],
        out_specs=[
            pl.BlockSpec((NBLK, D), lambda i: (i, 0)),
            pl.BlockSpec((NBLK, D), lambda i: (i, 0)),
            pl.BlockSpec((NBLK, D), lambda i: (i, 0)),
        ],
        out_shape=[jax.ShapeDtypeStruct((N, D), F32)] * 3,
    )(partials, partials, h, gru_wi, gru_wh, gru_bi, gru_bh, w_src, w_dst)


def _s2s_body(h_ref, nidx_ref, lwi_ref, lwh_ref, lb_ref,
              dw0_ref, dw1_ref, dw2_ref, db_ref,
              tp_ref, t1_ref, t2_ref,
              qstar, hs_s, cs_s, num_s, den_s, mx_s):
    qstar[...] = jnp.zeros((G, 2 * D), F32)
    hs_s[...] = jnp.zeros((G, D), F32)
    cs_s[...] = jnp.zeros((G, D), F32)
    giota = lax.broadcasted_iota(jnp.int32, (1, G), 1)
    nblocks = N // NBLK

    def step(_, carry):
        gates = (jnp.dot(qstar[...], lwi_ref[...],
                         precision=lax.Precision.DEFAULT,
                         preferred_element_type=F32)
                 + jnp.dot(hs_s[...], lwh_ref[...],
                           precision=lax.Precision.DEFAULT,
                           preferred_element_type=F32)
                 + lb_ref[...])
        i_g = jax.nn.sigmoid(gates[:, :D])
        f_g = jax.nn.sigmoid(gates[:, D:2 * D])
        g_g = jnp.tanh(gates[:, 2 * D:3 * D])
        o_g = jax.nn.sigmoid(gates[:, 3 * D:])
        cs = f_g * cs_s[...] + i_g * g_g
        hs = o_g * jnp.tanh(cs)
        cs_s[...] = cs
        hs_s[...] = hs

        mx_s[...] = jnp.full((1, G), -3e38, F32)

        def blk_max(b, c2):
            hb = h_ref[pl.ds(b * NBLK, NBLK), :]
            nb = nidx_ref[pl.ds(b * NBLK, NBLK), :]
            e_all = lax.dot_general(hb, hs, (((1,), (1,)), ((), ())),
                                    precision=HIGH,
                                    preferred_element_type=F32)
            onehot = nb == giota
            em = jnp.where(onehot, e_all, -3e38)
            mx_s[...] = jnp.maximum(mx_s[...], jnp.max(em, axis=0,
                                                       keepdims=True))
            return c2

        lax.fori_loop(0, nblocks, blk_max, 0, unroll=False)

        num_s[...] = jnp.zeros((G, D), F32)
        den_s[...] = jnp.zeros((G, 8), F32)
        mx = mx_s[...]

        def blk_sum(b, c2):
            hb = h_ref[pl.ds(b * NBLK, NBLK), :]
            nb = nidx_ref[pl.ds(b * NBLK, NBLK), :]
            e_all = lax.dot_general(hb, hs, (((1,), (1,)), ((), ())),
                                    precision=HIGH,
                                    preferred_element_type=F32)
            onehot = nb == giota
            a = jnp.where(onehot, jnp.exp(e_all - mx), 0.0)
            num_s[...] += lax.dot_general(a, hb, (((0,), (0,)), ((), ())),
                                          precision=HIGH,
                                          preferred_element_type=F32)
            ones_col = jnp.ones((NBLK, 8), F32)
            den_s[...] += lax.dot_general(a, ones_col, (((0,), (0,)), ((), ())),
                                          precision=HIGH,
                                          preferred_element_type=F32)
            return c2

        lax.fori_loop(0, nblocks, blk_sum, 0, unroll=False)
        den = den_s[:, :1]
        rvec = num_s[...] / jnp.where(den > 0.0, den, 1.0)
        qstar[:, :D] = hs
        qstar[:, D:] = rvec
        return carry

    lax.fori_loop(0, S2S_STEPS, step, 0, unroll=False)

    pool = qstar[...]
    h_g = h_ref[0:G, :]
    tp_ref[...] = jnp.dot(pool, dw0_ref[...],
                          precision=lax.Precision.DEFAULT,
                          preferred_element_type=F32) + db_ref[...]
    t1_ref[...] = jnp.dot(h_g, dw1_ref[...],
                          precision=lax.Precision.DEFAULT,
                          preferred_element_type=F32)
    t2_ref[...] = jnp.dot(h_g, dw2_ref[...],
                          precision=lax.Precision.DEFAULT,
                          preferred_element_type=F32)


def _tc_s2s(h, nidx2d, lstm_wi, lstm_wh, lstm_b, dw0, dw1, dw2, db):
    return pl.pallas_call(
        _s2s_body,
        in_specs=[pl.BlockSpec(memory_space=pltpu.VMEM)] * 9,
        out_specs=[pl.BlockSpec(memory_space=pltpu.VMEM)] * 3,
        out_shape=[jax.ShapeDtypeStruct((G, HID), F32)] * 3,
        scratch_shapes=[
            pltpu.VMEM((G, 2 * D), F32),
            pltpu.VMEM((G, D), F32),
            pltpu.VMEM((G, D), F32),
            pltpu.VMEM((G, D), F32),
            pltpu.VMEM((G, 8), F32),
            pltpu.VMEM((1, G), F32),
        ],
    )(h, nidx2d, lstm_wi, lstm_wh, lstm_b, dw0, dw1, dw2, db)


def _lane_gather(v, idx):
    dnums = lax.GatherDimensionNumbers(offset_dims=(),
                                       collapsed_slice_dims=(0,),
                                       start_index_map=(0,))
    return lax.gather(v, idx[:, None], dnums, (1,),
                      mode=lax.GatherScatterMode.PROMISE_IN_BOUNDS)


def _sc_couple_body(tp_hbm, t1_hbm, t2_hbm, b_hbm, a0_hbm, a1_hbm,
                    w_hbm, pb_hbm, out_hbm,
                    bv0, v00, v10, bv1, v01, v11,
                    buf_p0, buf_00, buf_10, buf_p1, buf_01, buf_11,
                    w_v, pb_v, outall,
                    sem_i0, sem_i1,
                    sem_p0, sem_00, sem_10, sem_p1, sem_01, sem_11,
                    sem_o):
    cid = lax.axis_index("c")
    sid = lax.axis_index("s")
    wid = sid * NC + cid
    lane = lax.broadcasted_iota(jnp.int32, (LANES,), 0)

    pltpu.sync_copy(w_hbm, w_v)
    pltpu.sync_copy(pb_hbm, pb_v)
    pb = pb_v[...]

    isets = ((bv0, v00, v10, sem_i0), (bv1, v01, v11, sem_i1))
    bsets = ((buf_p0, buf_00, buf_10, sem_p0, sem_00, sem_10),
             (buf_p1, buf_01, buf_11, sem_p1, sem_01, sem_11))

    def valid(k):
        return wid + k * NW < N_CCHUNK

    def prefetch_idx(s, k):
        bv, v0, v1, sem_i = isets[s]
        base = (wid + k * NW) * CK
        pltpu.async_copy(b_hbm.at[pl.ds(base, CK)], bv, sem_i)
        pltpu.async_copy(a0_hbm.at[pl.ds(base, CK)], v0, sem_i)
        pltpu.async_copy(a1_hbm.at[pl.ds(base, CK)], v1, sem_i)

    def wait_idx(s):
        bv, v0, v1, sem_i = isets[s]
        pltpu.make_async_copy(b_hbm.at[pl.ds(0, CK)], bv, sem_i).wait()
        pltpu.make_async_copy(a0_hbm.at[pl.ds(0, CK)], v0, sem_i).wait()
        pltpu.make_async_copy(a1_hbm.at[pl.ds(0, CK)], v1, sem_i).wait()

    def issue(s, k):
        bv, v0, v1, _ = isets[s]
        buf_p, buf_0, buf_1, sem_p, sem_0, sem_1 = bsets[s]
        pltpu.async_copy(tp_hbm.at[bv], buf_p, sem_p)
        pltpu.async_copy(t1_hbm.at[v0], buf_0, sem_0)
        pltpu.async_copy(t2_hbm.at[v1], buf_1, sem_1)

    def drain(s):
        bv, v0, v1, _ = isets[s]
        buf_p, buf_0, buf_1, sem_p, sem_0, sem_1 = bsets[s]
        pltpu.make_async_copy(tp_hbm.at[bv], buf_p, sem_p).wait()
        pltpu.make_async_copy(t1_hbm.at[v0], buf_0, sem_0).wait()
        pltpu.make_async_copy(t2_hbm.at[v1], buf_1, sem_1).wait()

    def out_slice(k):
        return out_hbm.at[pl.ds((wid + k * NW) * CK, CK)]

    def compute(s, k):
        buf_p, buf_0, buf_1, _, _, _ = bsets[s]

        def row(j, res):
            def col(kk, acc2):
                sl = pl.ds(kk * LANES, LANES)
                v = buf_p[j, sl] + buf_0[j, sl] + buf_1[j, sl]
                v = jnp.maximum(v, 0.0)
                return acc2 + v * w_v[sl]

            acc2 = lax.fori_loop(0, HID // LANES, col,
                                 jnp.zeros((LANES,), F32), unroll=8)
            for kk in (1, 2, 4, 8):
                acc2 = acc2 + _lane_gather(acc2, lane ^ kk)
            return jnp.where(lane == j, acc2, res)

        res = lax.fori_loop(0, CK, row, jnp.zeros((LANES,), F32),
                            unroll=False)
        val = (YR1 - YR0) / (1.0 + jnp.exp(-(res + pb))) + YR0
        outall[k] = val
        pltpu.async_copy(outall.at[k], out_slice(k), sem_o)

    def stage(s, k):
        s2 = 1 - s

        @pl.when(valid(k + 1))
        def _():
            wait_idx(s2)
            issue(s2, k + 1)

        @pl.when(valid(k))
        def _():
            drain(s)
            compute(s, k)

        @pl.when(valid(k + 2))
        def _():
            prefetch_idx(s, k + 2)

    prefetch_idx(0, 0)
    wait_idx(0)
    issue(0, 0)

    @pl.when(valid(1))
    def _():
        prefetch_idx(1, 1)

    def pair_body(p, carry):
        stage(0, 2 * p)
        stage(1, 2 * p + 1)
        return carry

    lax.fori_loop(0, CPAIRS, pair_body, 0, unroll=False)

    def drain_out(k, carry):
        @pl.when(valid(k))
        def _():
            pltpu.make_async_copy(outall.at[k], out_slice(k), sem_o).wait()

        return carry

    lax.fori_loop(0, CKIDX, drain_out, 0, unroll=False)


def _sc_couple(tp, t1, t2, bidx, a0, a1, w, pb):
    mesh = plsc.VectorSubcoreMesh(core_axis_name="c", subcore_axis_name="s")
    fn = pl.kernel(
        _sc_couple_body,
        out_type=jax.ShapeDtypeStruct((C,), F32),
        mesh=mesh,
        scratch_types=[
            pltpu.VMEM((CK,), jnp.int32),
            pltpu.VMEM((CK,), jnp.int32),
            pltpu.VMEM((CK,), jnp.int32),
            pltpu.VMEM((CK,), jnp.int32),
            pltpu.VMEM((CK,), jnp.int32),
            pltpu.VMEM((CK,), jnp.int32),
            pltpu.VMEM((CK, HID), F32),
            pltpu.VMEM((CK, HID), F32),
            pltpu.VMEM((CK, HID), F32),
            pltpu.VMEM((CK, HID), F32),
            pltpu.VMEM((CK, HID), F32),
            pltpu.VMEM((CK, HID), F32),
            pltpu.VMEM((HID,), F32),
            pltpu.VMEM((LANES,), F32),
            pltpu.VMEM((CKIDX, CK), F32),
            pltpu.SemaphoreType.DMA,
            pltpu.SemaphoreType.DMA,
            pltpu.SemaphoreType.DMA,
            pltpu.SemaphoreType.DMA,
            pltpu.SemaphoreType.DMA,
            pltpu.SemaphoreType.DMA,
            pltpu.SemaphoreType.DMA,
            pltpu.SemaphoreType.DMA,
            pltpu.SemaphoreType.DMA,
        ],
    )
    return fn(tp, t1, t2, bidx, a0, a1, w, pb)


def kernel(node, edge, edge_index, node_index, coupling_index, bond_type,
           x_atomic, W_pre, b_pre, msg_W, msg_b, gru_Wi, gru_Wh, gru_bi,
           gru_bh, lstm_Wi, lstm_Wh, lstm_b, dense_W, dense_b, pred_W,
           pred_b):
    w_src = msg_W[:D]
    w_dst = msg_W[D:2 * D]
    w_e = msg_W[2 * D:]
    b_pre2 = b_pre.reshape(1, D)
    msg_b2 = msg_b.reshape(1, D)
    gru_bi2 = gru_bi.reshape(1, 3 * D)
    gru_bh2 = gru_bh.reshape(1, 3 * D)
    lstm_b2 = lstm_b.reshape(1, 4 * D)
    dense_b2 = dense_b.reshape(1, HID)
    src = edge_index[0]
    dst = edge_index[1]
    nidx2d = node_index.reshape(N, 1)
    bidx = coupling_index[:, 3]
    a0 = coupling_index[:, 0]
    a1 = coupling_index[:, 1]
    zeros_tile = jnp.zeros((TILE_ROWS, D), F32)
    pred_w1 = pred_W.reshape(HID)
    pred_b16 = jnp.full((LANES,), pred_b[0], F32)

    h, hs, hd = _tc_pre(node, W_pre, b_pre2, w_src, w_dst)
    eb = _tc_eproj(edge, w_e, msg_b2)

    for _ in range(T_STEPS):
        partials = _sc_edge(hs, hd, eb, edge_index, zeros_tile)
        h, hs, hd = _tc_gru(partials, h, gru_Wi, gru_Wh,
                            gru_bi2, gru_bh2, w_src, w_dst)

    tp, t1, t2 = _tc_s2s(h, nidx2d, lstm_Wi, lstm_Wh, lstm_b2,
                         dense_W[:2 * D], dense_W[2 * D:3 * D],
                         dense_W[3 * D:], dense_b2)

    out = _sc_couple(tp, t1, t2, bidx, a0, a1, pred_w1, pred_b16)
    return out.reshape(C, 1)
